# Initial kernel scaffold; baseline (speedup 1.0000x reference)
import jax
import jax.numpy as jnp
from jax.experimental import pallas as pl


def kernel(adj_indices, adj_values, feature, tensor_x3, bing_rows, bing_cols, bing_values, W1, b1, W2, b2, W3, b3, W4, b4):
    raise NotImplementedError("write your pallas kernel here")


# speedup vs baseline: 1.1805x; 1.1805x over previous
import dataclasses
import functools

import jax
import jax.numpy as jnp
from jax import lax
from jax.experimental import pallas as pl
from jax.experimental.pallas import tpu as pltpu
from jax.experimental.pallas import tpu_sc as plsc

_N = 50000
_E = 800000
_M = 200000
_EB = 200000

_NP = 50048
_EP = 819200
_EBP = 212992
_MR = 66688
_MRP = 66816

_B = 1024


def _compiler_params():
    return pltpu.CompilerParams(use_tc_tiling_on_sc=False,
                                needs_layout_passes=False)


def _scale_rows(g_ref, vals_ref, ncols, nedges):
    @pl.loop(0, nedges // 16)
    def _(grp):
        base = grp * 16
        vv = vals_ref[pl.ds(base, 16)]
        ridx = base + lax.iota(jnp.int32, 16)
        for c in range(ncols):
            cidx = jnp.full((16,), c, jnp.int32)
            colv = plsc.load_gather(g_ref, [ridx, cidx])
            plsc.store_scatter(g_ref, [ridx, cidx], colv * vv)


def _sc_mesh():
    return plsc.VectorSubcoreMesh(core_axis_name="c", subcore_axis_name="s")


def _writeback(t, acc, out_ref, nrows, dst_base=0):
    w = ((nrows // 16) + 7) // 8 * 8
    rem = nrows - 15 * w

    @pl.when(t < 15)
    def _():
        pltpu.sync_copy(acc.at[pl.ds(t * w, w)],
                        out_ref.at[pl.ds(dst_base + t * w, w)])

    @pl.when(t == 15)
    def _():
        pltpu.sync_copy(acc.at[pl.ds(15 * w, rem)],
                        out_ref.at[pl.ds(dst_base + 15 * w, rem)])


def _sc_spmm1(fp, cols, rows, vals, z1):
    quota = _EP // 32
    nblk = quota // _B

    @functools.partial(
        pl.kernel,
        out_type=jax.ShapeDtypeStruct((2, _N, 16), jnp.float32),
        mesh=_sc_mesh(),
        scratch_types=[
            pltpu.VMEM((_B,), jnp.int32),
            pltpu.VMEM((_B,), jnp.int32),
            pltpu.VMEM((_B,), jnp.float32),
            pltpu.VMEM((_B, 16), jnp.float32),
            pltpu.VMEM_SHARED((_NP, 16), jnp.float32),
        ],
        compiler_params=_compiler_params(),
    )
    def k(fp_hbm, cols_hbm, rows_hbm, vals_hbm, z_hbm, out_hbm,
          cols_v, rows_v, vals_v, g, acc):
        c = lax.axis_index("c")
        t = lax.axis_index("s")
        zrows = _NP // 16
        pltpu.sync_copy(z_hbm.at[pl.ds(t * zrows, zrows)],
                        acc.at[pl.ds(t * zrows, zrows)])
        plsc.subcore_barrier()

        base_e = (c * 16 + t) * quota

        @pl.loop(0, nblk)
        def _(blk):
            off = base_e + blk * _B
            pltpu.sync_copy(cols_hbm.at[pl.ds(off, _B)], cols_v)
            pltpu.sync_copy(rows_hbm.at[pl.ds(off, _B)], rows_v)
            pltpu.sync_copy(vals_hbm.at[pl.ds(off, _B)], vals_v)
            pltpu.sync_copy(fp_hbm.at[cols_v], g)
            _scale_rows(g, vals_v, 16, _B)
            pltpu.sync_copy(g, acc.at[rows_v], add=True)

        plsc.subcore_barrier()

        @pl.when(c == 0)
        def _():
            _writeback(t, acc, out_hbm.at[0], _N)

        @pl.when(c == 1)
        def _():
            _writeback(t, acc, out_hbm.at[1], _N)

    return k(fp, cols, rows, vals, z1)


def _sc_spmm2(xc, cols, rows, vals, z2):
    quota = _EP // 16
    nblk = quota // _B

    @functools.partial(
        pl.kernel,
        out_type=jax.ShapeDtypeStruct((8, _N, 16), jnp.float32),
        mesh=_sc_mesh(),
        scratch_types=[
            pltpu.VMEM((_B,), jnp.int32),
            pltpu.VMEM((_B,), jnp.int32),
            pltpu.VMEM((_B,), jnp.float32),
            pltpu.VMEM((_B, 16), jnp.float32),
            pltpu.VMEM_SHARED((_NP, 16), jnp.float32),
        ],
        compiler_params=_compiler_params(),
    )
    def k(xc_hbm, cols_hbm, rows_hbm, vals_hbm, z_hbm, out_hbm,
          cols_v, rows_v, vals_v, g, acc):
        c = lax.axis_index("c")
        t = lax.axis_index("s")
        zrows = _NP // 16

        def one_chunk(kk):
            pltpu.sync_copy(z_hbm.at[pl.ds(t * zrows, zrows)],
                            acc.at[pl.ds(t * zrows, zrows)])
            plsc.subcore_barrier()

            @pl.loop(0, nblk)
            def _(blk):
                off = t * quota + blk * _B
                pltpu.sync_copy(cols_hbm.at[pl.ds(off, _B)], cols_v)
                pltpu.sync_copy(rows_hbm.at[pl.ds(off, _B)], rows_v)
                pltpu.sync_copy(vals_hbm.at[pl.ds(off, _B)], vals_v)
                pltpu.sync_copy(xc_hbm.at[kk].at[cols_v], g)
                _scale_rows(g, vals_v, 16, _B)
                pltpu.sync_copy(g, acc.at[rows_v], add=True)

            plsc.subcore_barrier()
            _writeback(t, acc, out_hbm.at[kk], _N)
            plsc.subcore_barrier()

        @pl.loop(0, 4)
        def _(j):
            one_chunk(c * 4 + j)

    return k(xc, cols, rows, vals, z2)


def _sc_spmm3(xc, cols, rows, vals, z3):
    quota = _EBP // 16
    nblk = quota // _B

    @functools.partial(
        pl.kernel,
        out_type=jax.ShapeDtypeStruct((16, _M, 16), jnp.float32),
        mesh=_sc_mesh(),
        scratch_types=[
            pltpu.VMEM((_B,), jnp.int32),
            pltpu.VMEM((_B,), jnp.int32),
            pltpu.VMEM((_B,), jnp.int32),
            pltpu.VMEM((_B,), jnp.float32),
            pltpu.VMEM((_B, 16), jnp.float32),
            pltpu.VMEM_SHARED((_MRP, 16), jnp.float32),
        ],
        compiler_params=_compiler_params(),
    )
    def k(xc_hbm, cols_hbm, rows_hbm, vals_hbm, z_hbm, out_hbm,
          cols_v, rows_v, ridx_v, vals_v, g, acc):
        c = lax.axis_index("c")
        t = lax.axis_index("s")
        zrows = _MRP // 16

        def one_pass(kk, h, nrows):
            pltpu.sync_copy(z_hbm.at[pl.ds(t * zrows, zrows)],
                            acc.at[pl.ds(t * zrows, zrows)])
            plsc.subcore_barrier()

            @pl.loop(0, nblk)
            def _(blk):
                off = t * quota + blk * _B
                pltpu.sync_copy(cols_hbm.at[pl.ds(off, _B)], cols_v)
                pltpu.sync_copy(rows_hbm.at[pl.ds(off, _B)], rows_v)
                pltpu.sync_copy(vals_hbm.at[pl.ds(off, _B)], vals_v)
                pltpu.sync_copy(xc_hbm.at[kk].at[cols_v], g)
                _scale_rows(g, vals_v, 16, _B)

                @pl.loop(0, _B // 16)
                def _(grp):
                    sl = pl.ds(grp * 16, 16)
                    rv = rows_v[sl] - h * _MR
                    ok = (rv >= 0) & (rv < _MR)
                    ridx_v[sl] = jnp.where(ok, rv, _MR + lax.iota(jnp.int32, 16))

                pltpu.sync_copy(g, acc.at[ridx_v], add=True)

            plsc.subcore_barrier()
            _writeback(t, acc, out_hbm.at[kk], nrows, dst_base=h * _MR)
            plsc.subcore_barrier()

        for h in range(3):
            nrows = _MR if h < 2 else _M - 2 * _MR

            @pl.loop(0, 8)
            def _(j, h=h, nrows=nrows):
                one_pass(c * 8 + j, h, nrows)

    return k(xc, cols, rows, vals, z3)


def _tc_layer1(s1pair, W1, b1, W2b, b2):
    nb = 2000
    grid = _N // nb

    def body(sa_ref, sb_ref, w1_ref, b1_ref, w2b_ref, b2_ref, h1r_ref, g2b_ref):
        x = (sa_ref[...] + sb_ref[...])[:, :4]
        h1r_ref[...] = jnp.maximum(
            jnp.dot(x, w1_ref[...], preferred_element_type=jnp.float32)
            + b1_ref[...], 0.0)
        g2b_ref[...] = (
            jnp.dot(x, w2b_ref[...], preferred_element_type=jnp.float32)
            + b2_ref[...])

    return pl.pallas_call(
        body,
        grid=(grid,),
        in_specs=[
            pl.BlockSpec((nb, 16), lambda i: (i, 0)),
            pl.BlockSpec((nb, 16), lambda i: (i, 0)),
            pl.BlockSpec((4, 128), lambda i: (0, 0)),
            pl.BlockSpec((1, 128), lambda i: (0, 0)),
            pl.BlockSpec((4, 256), lambda i: (0, 0)),
            pl.BlockSpec((1, 256), lambda i: (0, 0)),
        ],
        out_specs=[
            pl.BlockSpec((nb, 128), lambda i: (i, 0)),
            pl.BlockSpec((nb, 256), lambda i: (i, 0)),
        ],
        out_shape=[
            jax.ShapeDtypeStruct((_N, 128), jnp.float32),
            jax.ShapeDtypeStruct((_N, 256), jnp.float32),
        ],
    )(s1pair[0], s1pair[1], W1, b1, W2b, b2)


def _tc_layer2(s2c, g2b, W2a):
    nb = 2000
    grid = _N // nb

    def body(s2_ref, g2b_ref, w_ref, h2_ref):
        x = jnp.concatenate([s2_ref[j] for j in range(8)], axis=1)
        h2_ref[...] = jnp.maximum(
            jnp.dot(x, w_ref[...], preferred_element_type=jnp.float32)
            + g2b_ref[...], 0.0)

    return pl.pallas_call(
        body,
        grid=(grid,),
        in_specs=[
            pl.BlockSpec((8, nb, 16), lambda i: (0, i, 0)),
            pl.BlockSpec((nb, 256), lambda i: (i, 0)),
            pl.BlockSpec((128, 256), lambda i: (0, 0)),
        ],
        out_specs=pl.BlockSpec((nb, 256), lambda i: (i, 0)),
        out_shape=jax.ShapeDtypeStruct((_N, 256), jnp.float32),
    )(s2c, g2b, W2a)


def _tc_layer34(s3, x3, W3a, W3b, b3, W4a, W4b, b4):
    mb = 2000
    grid = _M // mb

    def body(s3_ref, x3_ref, w3a_ref, w3b_ref, b3_ref, w4a_ref, w4b_ref,
             b4_ref, h3_ref, h4_ref):
        x3v = x3_ref[...]
        h3 = jnp.maximum(
            jnp.dot(s3_ref[...], w3a_ref[...], preferred_element_type=jnp.float32)
            + jnp.dot(x3v, w3b_ref[...], preferred_element_type=jnp.float32)
            + b3_ref[...], 0.0)
        h3_ref[...] = h3
        h4_ref[...] = (
            jnp.dot(h3, w4a_ref[...], preferred_element_type=jnp.float32)
            + jnp.dot(x3v, w4b_ref[...], preferred_element_type=jnp.float32)
            + b4_ref[...])

    return pl.pallas_call(
        body,
        grid=(grid,),
        in_specs=[
            pl.BlockSpec((mb, 256), lambda i: (i, 0)),
            pl.BlockSpec((mb, 4), lambda i: (i, 0)),
            pl.BlockSpec((256, 512), lambda i: (0, 0)),
            pl.BlockSpec((4, 512), lambda i: (0, 0)),
            pl.BlockSpec((1, 512), lambda i: (0, 0)),
            pl.BlockSpec((512, 4), lambda i: (0, 0)),
            pl.BlockSpec((4, 4), lambda i: (0, 0)),
            pl.BlockSpec((1, 4), lambda i: (0, 0)),
        ],
        out_specs=[
            pl.BlockSpec((mb, 512), lambda i: (i, 0)),
            pl.BlockSpec((mb, 4), lambda i: (i, 0)),
        ],
        out_shape=[
            jax.ShapeDtypeStruct((_M, 512), jnp.float32),
            jax.ShapeDtypeStruct((_M, 4), jnp.float32),
        ],
    )(s3, x3, W3a, W3b, b3, W4a, W4b, b4)


def kernel(adj_indices, adj_values, feature, tensor_x3, bing_rows, bing_cols,
           bing_values, W1, b1, W2, b2, W3, b3, W4, b4):
    f32 = jnp.float32
    rows = adj_indices[0].astype(jnp.int32)
    cols = adj_indices[1].astype(jnp.int32)
    vals = adj_values.astype(f32)

    padn = _EP - _E
    ar = jnp.arange(padn, dtype=jnp.int32)
    rows_p = jnp.concatenate([rows, _N + (ar % 16)])
    cols_p = jnp.concatenate([cols, ar % 2048])
    vals_p = jnp.concatenate([vals, jnp.zeros((padn,), f32)])

    padb = _EBP - _EB
    ab = jnp.arange(padb, dtype=jnp.int32)
    brows_p = jnp.concatenate([bing_rows.astype(jnp.int32),
                               jnp.full((padb,), 1 << 20, jnp.int32)])
    bcols_p = jnp.concatenate([bing_cols.astype(jnp.int32), ab % 2048])
    bvals_p = jnp.concatenate([bing_values.astype(f32), jnp.zeros((padb,), f32)])

    fp = jnp.pad(feature, ((0, 0), (0, 12)))

    z1 = jnp.zeros((_NP, 16), f32)
    z3 = jnp.zeros((_MRP, 16), f32)

    b1r = b1.reshape(1, 128)
    b2r = b2.reshape(1, 256)
    b3r = b3.reshape(1, 512)
    b4r = b4.reshape(1, 4)
    W2a, W2b = W2[:128], W2[128:]
    W3a, W3b = W3[:256], W3[256:]
    W4a, W4b = W4[:512], W4[512:]

    s1pair = _sc_spmm1(fp, cols_p, rows_p, vals_p, z1)
    h1r, g2b = _tc_layer1(s1pair, W1, b1r, W2b, b2r)

    xc2 = h1r.reshape(_N, 8, 16).transpose(1, 0, 2)
    s2c = _sc_spmm2(xc2, cols_p, rows_p, vals_p, z1)
    h2 = _tc_layer2(s2c, g2b, W2a)

    xc3 = h2.reshape(_N, 16, 16).transpose(1, 0, 2)
    s3c = _sc_spmm3(xc3, bcols_p, brows_p, bvals_p, z3)
    s3 = s3c.transpose(1, 0, 2).reshape(_M, 256)

    h3, h4 = _tc_layer34(s3, tensor_x3, W3a, W3b, b3r, W4a, W4b, b4r)

    h1 = jnp.concatenate([h1r, feature], axis=1)
    h3c = jnp.concatenate([h3, tensor_x3], axis=1)
    return (h4, h1, h3, h3c, h4)
